# baseline (speedup 1.0000x reference)
import functools

import jax
import jax.numpy as jnp
from jax import lax
from jax.experimental import pallas as pl
from jax.experimental.pallas import tpu as pltpu
from jax.experimental.pallas import tpu_sc as plsc

_B = 16384
_D = 64
_N_FEAT = 26
_FPT = 13
_OC = _FPT * _D

_PERMUTES = tuple(
    (i // _FPT, i % 2, (i % _FPT) * _D, (i // 2) * _D) for i in range(_N_FEAT)
)

_INFO = plsc.get_sparse_core_info()
_NC = _INFO.num_cores
_NS = _INFO.num_subcores
_NW = _NC * _NS
_CW = _B // _NW

_mesh = plsc.VectorSubcoreMesh(core_axis_name="c", subcore_axis_name="s")


@functools.partial(
    pl.kernel,
    mesh=_mesh,
    compiler_params=pltpu.CompilerParams(use_tc_tiling_on_sc=True),
    out_type=(
        jax.ShapeDtypeStruct((_OC, _B), jnp.float32),
        jax.ShapeDtypeStruct((_OC, _B), jnp.float32),
    ),
    scratch_types=([pltpu.SemaphoreType.DMA]),
)
def _permute_sc(v0, v1, o0, o1, sem):
    ins = (v0, v1)
    outs = (o0, o1)

    wid = lax.axis_index("s") * _NC + lax.axis_index("c")
    cols = pl.ds(wid * _CW, _CW)

    def copy(f):
        ii, oi, istart, ostart = _PERMUTES[f]
        return pltpu.make_async_copy(
            ins[ii].at[pl.ds(istart, _D), cols],
            outs[oi].at[pl.ds(ostart, _D), cols],
            sem,
        )

    for f in range(_N_FEAT):
        copy(f).start()
    for f in range(_N_FEAT):
        copy(f).wait()


@jax.jit
def kernel(values_0, values_1):
    o0t, o1t = _permute_sc(values_0.T, values_1.T)
    return o0t.T, o1t.T


# speedup vs baseline: 22.1298x; 22.1298x over previous
import functools

import jax
import jax.numpy as jnp
from jax import lax
from jax.experimental import pallas as pl
from jax.experimental.pallas import tpu as pltpu
from jax.experimental.pallas import tpu_sc as plsc

_B = 16384
_D = 64
_N_FEAT = 26
_FPT = 13
_OC = _FPT * _D

_PERMUTES = tuple(
    (i // _FPT, i % 2, (i % _FPT) * _D, (i // 2) * _D) for i in range(_N_FEAT)
)
_SC_CHUNKS = tuple(p for p in _PERMUTES if p[1] == 0)
_NSC = len(_SC_CHUNKS)

_INFO = plsc.get_sparse_core_info()
_NC = _INFO.num_cores
_NS = _INFO.num_subcores
_NW = _NC * _NS
_CW = _B // _NW

_NBUF = 3

_mesh = plsc.VectorSubcoreMesh(core_axis_name="c", subcore_axis_name="s")


@functools.partial(
    pl.kernel,
    mesh=_mesh,
    compiler_params=pltpu.CompilerParams(use_tc_tiling_on_sc=True),
    out_type=jax.ShapeDtypeStruct((_OC, _B), jnp.float32),
    scratch_types=(
        [pltpu.VMEM((_D, _CW), jnp.float32) for _ in range(_NBUF)]
        + [pltpu.SemaphoreType.DMA for _ in range(2 * _NBUF)]
    ),
)
def _permute_sc(v0, v1, o0, buf0, buf1, buf2, sg0, sg1, sg2, ss0, ss1, ss2):
    bufs = (buf0, buf1, buf2)
    sem_g = (sg0, sg1, sg2)
    sem_s = (ss0, ss1, ss2)
    ins = (v0, v1)

    wid = lax.axis_index("s") * _NC + lax.axis_index("c")
    cols = pl.ds(wid * _CW, _CW)

    def g_copy(f, s):
        ii, _, istart, _ = _SC_CHUNKS[f]
        return pltpu.make_async_copy(
            ins[ii].at[pl.ds(istart, _D), cols], bufs[s], sem_g[s]
        )

    def s_copy(f, s):
        _, _, _, ostart = _SC_CHUNKS[f]
        return pltpu.make_async_copy(
            bufs[s], o0.at[pl.ds(ostart, _D), cols], sem_s[s]
        )

    g_copy(0, 0).start()
    g_copy(1, 1).start()
    for f in range(_NSC):
        s = f % _NBUF
        g_copy(f, s).wait()
        s_copy(f, s).start()
        if f + 2 < _NSC:
            nxt = (f + 2) % _NBUF
            if f - 1 >= 0:
                s_copy(f - 1, nxt).wait()
            g_copy(f + 2, nxt).start()
    s_copy(_NSC - 2, (_NSC - 2) % _NBUF).wait()
    s_copy(_NSC - 1, (_NSC - 1) % _NBUF).wait()


_CB = 2048


def _tc_copy_body(i_ref, o_ref):
    o_ref[...] = i_ref[...]


def _tc_chunk_copy(v, nchunks, in_block):
    return pl.pallas_call(
        _tc_copy_body,
        grid=(nchunks, _B // _CB),
        in_specs=[pl.BlockSpec((_D, _CB), lambda j, c: (in_block(j), c))],
        out_specs=pl.BlockSpec((_D, _CB), lambda j, c: (j, c)),
        out_shape=jax.ShapeDtypeStruct((nchunks * _D, _B), jnp.float32),
        compiler_params=pltpu.CompilerParams(
            dimension_semantics=("parallel", "parallel")
        ),
    )(v)


@jax.jit
def kernel(values_0, values_1):
    v0t = values_0.T
    v1t = values_1.T
    o0t = _permute_sc(v0t, v1t)
    o1a = _tc_chunk_copy(v0t, 6, lambda j: 2 * j + 1)
    o1b = _tc_chunk_copy(v1t, 7, lambda j: 2 * j)
    o1t = jnp.concatenate([o1a, o1b], axis=0)
    return o0t.T, o1t.T


# speedup vs baseline: 28.0904x; 1.2693x over previous
import functools

import jax
import jax.numpy as jnp
from jax import lax
from jax.experimental import pallas as pl
from jax.experimental.pallas import tpu as pltpu
from jax.experimental.pallas import tpu_sc as plsc

_B = 16384
_D = 64
_N_FEAT = 26
_FPT = 13
_OC = _FPT * _D

_PERMUTES = tuple(
    (i // _FPT, i % 2, (i % _FPT) * _D, (i // 2) * _D) for i in range(_N_FEAT)
)
_SC_CHUNKS = tuple(p for p in _PERMUTES if p[1] == 0)
_NSC = len(_SC_CHUNKS)

_INFO = plsc.get_sparse_core_info()
_NC = _INFO.num_cores
_NS = _INFO.num_subcores
_NW = _NC * _NS
_CW = _B // _NW

_NBUF = 3

_mesh = plsc.VectorSubcoreMesh(core_axis_name="c", subcore_axis_name="s")


@functools.partial(
    pl.kernel,
    mesh=_mesh,
    compiler_params=pltpu.CompilerParams(use_tc_tiling_on_sc=True),
    out_type=jax.ShapeDtypeStruct((_OC, _B), jnp.float32),
    scratch_types=(
        [pltpu.VMEM((_D, _CW), jnp.float32) for _ in range(_NBUF)]
        + [pltpu.SemaphoreType.DMA for _ in range(2 * _NBUF)]
    ),
)
def _permute_sc(v0, v1, o0, buf0, buf1, buf2, sg0, sg1, sg2, ss0, ss1, ss2):
    bufs = (buf0, buf1, buf2)
    sem_g = (sg0, sg1, sg2)
    sem_s = (ss0, ss1, ss2)
    ins = (v0, v1)

    wid = lax.axis_index("s") * _NC + lax.axis_index("c")
    cols = pl.ds(wid * _CW, _CW)

    def g_copy(f, s):
        ii, _, istart, _ = _SC_CHUNKS[f]
        return pltpu.make_async_copy(
            ins[ii].at[pl.ds(istart, _D), cols], bufs[s], sem_g[s]
        )

    def s_copy(f, s):
        _, _, _, ostart = _SC_CHUNKS[f]
        return pltpu.make_async_copy(
            bufs[s], o0.at[pl.ds(ostart, _D), cols], sem_s[s]
        )

    g_copy(0, 0).start()
    g_copy(1, 1).start()
    for f in range(_NSC):
        s = f % _NBUF
        g_copy(f, s).wait()
        s_copy(f, s).start()
        if f + 2 < _NSC:
            nxt = (f + 2) % _NBUF
            if f - 1 >= 0:
                s_copy(f - 1, nxt).wait()
            g_copy(f + 2, nxt).start()
    s_copy(_NSC - 2, (_NSC - 2) % _NBUF).wait()
    s_copy(_NSC - 1, (_NSC - 1) % _NBUF).wait()


_CB = 2048


def _tc_body(v0_ref, v1_ref, o_ref):
    j = pl.program_id(1)

    @pl.when(j < 6)
    def _():
        o_ref[...] = v0_ref[...]

    @pl.when(j >= 6)
    def _():
        o_ref[...] = v1_ref[...]


def _tc_out1(v0t, v1t):
    return pl.pallas_call(
        _tc_body,
        grid=(_B // _CB, _FPT),
        in_specs=[
            pl.BlockSpec(
                (_D, _CB), lambda c, j: (jnp.where(j < 6, 2 * j + 1, 11), c)
            ),
            pl.BlockSpec(
                (_D, _CB), lambda c, j: (jnp.where(j < 6, 0, 2 * (j - 6)), c)
            ),
        ],
        out_specs=pl.BlockSpec((_D, _CB), lambda c, j: (j, c)),
        out_shape=jax.ShapeDtypeStruct((_OC, _B), jnp.float32),
        compiler_params=pltpu.CompilerParams(
            dimension_semantics=("arbitrary", "arbitrary")
        ),
    )(v0t, v1t)


@jax.jit
def kernel(values_0, values_1):
    v0t = values_0.T
    v1t = values_1.T
    o0t = _permute_sc(v0t, v1t)
    o1t = _tc_out1(v0t, v1t)
    return o0t.T, o1t.T


# speedup vs baseline: 37.7500x; 1.3439x over previous
import functools

import jax
import jax.numpy as jnp
from jax import lax
from jax.experimental import pallas as pl
from jax.experimental.pallas import tpu as pltpu
from jax.experimental.pallas import tpu_sc as plsc

_B = 16384
_D = 64
_N_FEAT = 26
_FPT = 13
_OC = _FPT * _D

_PERMUTES = tuple(
    (i // _FPT, i % 2, (i % _FPT) * _D, (i // 2) * _D) for i in range(_N_FEAT)
)
_SC_CHUNKS = tuple(p for p in _PERMUTES if p[1] == 0)
_NSC = len(_SC_CHUNKS)

_INFO = plsc.get_sparse_core_info()
_NC = _INFO.num_cores
_NS = _INFO.num_subcores
_NW = _NC * _NS
_CW = _B // _NW

_NBUF = 3

_mesh = plsc.VectorSubcoreMesh(core_axis_name="c", subcore_axis_name="s")


@functools.partial(
    pl.kernel,
    mesh=_mesh,
    compiler_params=pltpu.CompilerParams(use_tc_tiling_on_sc=True),
    out_type=jax.ShapeDtypeStruct((_OC, _B), jnp.float32),
    scratch_types=(
        [pltpu.VMEM((_D, _CW), jnp.float32) for _ in range(_NBUF)]
        + [pltpu.SemaphoreType.DMA for _ in range(2 * _NBUF)]
    ),
)
def _permute_sc(v0, v1, o0, buf0, buf1, buf2, sg0, sg1, sg2, ss0, ss1, ss2):
    bufs = (buf0, buf1, buf2)
    sem_g = (sg0, sg1, sg2)
    sem_s = (ss0, ss1, ss2)
    ins = (v0, v1)

    wid = lax.axis_index("s") * _NC + lax.axis_index("c")
    cols = pl.ds(wid * _CW, _CW)

    def g_copy(f, s):
        ii, _, istart, _ = _SC_CHUNKS[f]
        return pltpu.make_async_copy(
            ins[ii].at[pl.ds(istart, _D), cols], bufs[s], sem_g[s]
        )

    def s_copy(f, s):
        _, _, _, ostart = _SC_CHUNKS[f]
        return pltpu.make_async_copy(
            bufs[s], o0.at[pl.ds(ostart, _D), cols], sem_s[s]
        )

    g_copy(0, 0).start()
    g_copy(1, 1).start()
    for f in range(_NSC):
        s = f % _NBUF
        g_copy(f, s).wait()
        s_copy(f, s).start()
        if f + 2 < _NSC:
            nxt = (f + 2) % _NBUF
            if f - 1 >= 0:
                s_copy(f - 1, nxt).wait()
            g_copy(f + 2, nxt).start()
    s_copy(_NSC - 2, (_NSC - 2) % _NBUF).wait()
    s_copy(_NSC - 1, (_NSC - 1) % _NBUF).wait()


def _tc_body(v0_ref, v1_ref, o_ref):
    j = pl.program_id(0)

    @pl.when(j < 6)
    def _():
        o_ref[...] = v0_ref[...]

    @pl.when(j >= 6)
    def _():
        o_ref[...] = v1_ref[...]


def _tc_out1(v0t, v1t):
    return pl.pallas_call(
        _tc_body,
        grid=(_FPT,),
        in_specs=[
            pl.BlockSpec(
                (_D, _B), lambda j: (jnp.where(j < 6, 2 * j + 1, 11), 0)
            ),
            pl.BlockSpec(
                (_D, _B), lambda j: (jnp.where(j < 6, 0, 2 * (j - 6)), 0)
            ),
        ],
        out_specs=pl.BlockSpec((_D, _B), lambda j: (j, 0)),
        out_shape=jax.ShapeDtypeStruct((_OC, _B), jnp.float32),
        compiler_params=pltpu.CompilerParams(
            dimension_semantics=("arbitrary",)
        ),
    )(v0t, v1t)


@jax.jit
def kernel(values_0, values_1):
    v0t = values_0.T
    v1t = values_1.T
    o0t = _permute_sc(v0t, v1t)
    o1t = _tc_out1(v0t, v1t)
    return o0t.T, o1t.T
